# Initial kernel scaffold; baseline (speedup 1.0000x reference)
import jax
import jax.numpy as jnp
from jax.experimental import pallas as pl


def kernel(defect_location, defect_sites, ricci_flow_rate, cohesion, friction_angle):
    raise NotImplementedError("write your pallas kernel here")


# speedup vs baseline: 18.6075x; 18.6075x over previous
import functools

import jax
import jax.numpy as jnp
from jax import lax
from jax.experimental import pallas as pl
from jax.experimental.pallas import tpu as pltpu
from jax.experimental.pallas import tpu_sc as plsc

Qn = 1024
Kn = 100000
Dn = 16
KT = 2048
KPAD = 100352
NT = KPAD // KT
PACK = 128 // Dn
NW = 32
BPW = (2 * Qn) // NW

_BIG_I32 = 2**30


def _scan_kernel(s_ref, xt_ref, i1_ref, i2_ref, b1v, b1i, b2v, b2i):
    k = pl.program_id(0)

    @pl.when(k == 0)
    def _init():
        b1v[...] = jnp.full((1, Qn), jnp.inf, jnp.float32)
        b2v[...] = jnp.full((1, Qn), jnp.inf, jnp.float32)
        b1i[...] = jnp.zeros((1, Qn), jnp.int32)
        b2i[...] = jnp.zeros((1, Qn), jnp.int32)

    s = s_ref[...]
    colnorm = jnp.sum(s * s, axis=1, keepdims=True)
    xt = xt_ref[...]
    xs = lax.dot_general(
        s, xt, (((1,), (0,)), ((), ())),
        preferred_element_type=jnp.float32,
        precision=lax.Precision.HIGHEST)
    scores = colnorm - (xs + xs)

    tmin = jnp.min(scores, axis=0, keepdims=True)
    row = lax.broadcasted_iota(jnp.int32, (KT, Qn), 0)
    cand = jnp.min(jnp.where(scores == tmin, row, _BIG_I32),
                   axis=0, keepdims=True) + k * KT

    b1v_o, b1i_o = b1v[...], b1i[...]
    b2v_o, b2i_o = b2v[...], b2i[...]
    better1 = tmin < b1v_o
    better2 = tmin < b2v_o
    b1v[...] = jnp.where(better1, tmin, b1v_o)
    b1i[...] = jnp.where(better1, cand, b1i_o)
    b2v[...] = jnp.where(better1, b1v_o, jnp.where(better2, tmin, b2v_o))
    b2i[...] = jnp.where(better1, b1i_o, jnp.where(better2, cand, b2i_o))

    @pl.when(k == NT - 1)
    def _fin():
        i1_ref[...] = b1i[...]
        i2_ref[...] = b2i[...]


_scan = pl.pallas_call(
    _scan_kernel,
    grid=(NT,),
    in_specs=[
        pl.BlockSpec((KT, Dn), lambda k: (k, 0)),
        pl.BlockSpec((Dn, Qn), lambda k: (0, 0)),
    ],
    out_specs=[
        pl.BlockSpec((1, Qn), lambda k: (0, 0)),
        pl.BlockSpec((1, Qn), lambda k: (0, 0)),
    ],
    out_shape=[
        jax.ShapeDtypeStruct((1, Qn), jnp.int32),
        jax.ShapeDtypeStruct((1, Qn), jnp.int32),
    ],
    scratch_shapes=[
        pltpu.VMEM((1, Qn), jnp.float32),
        pltpu.VMEM((1, Qn), jnp.int32),
        pltpu.VMEM((1, Qn), jnp.float32),
        pltpu.VMEM((1, Qn), jnp.int32),
    ],
)


@functools.cache
def _make_sc_gather():
    @functools.partial(
        pl.kernel,
        mesh=plsc.VectorSubcoreMesh(core_axis_name="c", subcore_axis_name="s"),
        out_type=jax.ShapeDtypeStruct((2 * Qn, 8 * Dn), jnp.float32),
        scratch_types=[
            pltpu.VMEM((BPW,), jnp.int32),
            pltpu.VMEM((BPW, 8 * Dn), jnp.float32),
            pltpu.SemaphoreType.DMA,
        ],
    )
    def _sc_gather(rows_hbm, idx_hbm, out_hbm, idx_v, rows_v, sem):
        wid = lax.axis_index("s") * 2 + lax.axis_index("c")
        base = wid * BPW
        pltpu.sync_copy(idx_hbm.at[pl.ds(base, BPW)], idx_v)
        pltpu.async_copy(rows_hbm.at[idx_v], rows_v, sem).wait()
        pltpu.sync_copy(rows_v, out_hbm.at[pl.ds(base, BPW)])

    return _sc_gather


def _extract(r, sub):
    lane_grp = lax.broadcasted_iota(jnp.int32, (Qn, PACK * Dn), 1) // Dn
    g = jnp.where(lane_grp == sub, r, 0.0)
    acc = g[:, 0:Dn]
    for c in range(1, PACK):
        acc = acc + g[:, c * Dn:(c + 1) * Dn]
    return acc


def _epi_kernel(x_ref, rows_ref, i1_ref, i2_ref, scal_ref, out_ref):
    x = x_ref[...]
    i1 = i1_ref[...]
    i2 = i2_ref[...]
    s1 = _extract(rows_ref[0], i1 % PACK)
    s2 = _extract(rows_ref[1], i2 % PACK)
    rate = scal_ref[0]
    cohesion = scal_ref[1]
    tanfa = scal_ref[2]

    diff1 = x - s1
    diff2 = x - s2
    d1 = jnp.sqrt(jnp.sum(diff1 * diff1, axis=1, keepdims=True))
    d2 = jnp.sqrt(jnp.sum(diff2 * diff2, axis=1, keepdims=True))
    pick1 = (d1 < d2) | ((d1 == d2) & (i1 < i2))
    sw = jnp.where(pick1, s1, s2)

    ricci = rate * (sw - x)
    mag = jnp.sqrt(jnp.sum(ricci * ricci, axis=1, keepdims=True))
    xnorm = jnp.sqrt(jnp.sum(x * x, axis=1, keepdims=True))
    normal = jnp.abs(jnp.sum(x * ricci, axis=1, keepdims=True)) / (xnorm + 1e-8)
    thresh = cohesion + normal * tanfa
    exceeds = mag > thresh
    out_ref[...] = jnp.where(exceeds, ricci * 2.0, ricci * 0.5)


_epi = pl.pallas_call(
    _epi_kernel,
    in_specs=[
        pl.BlockSpec((Qn, Dn), lambda: (0, 0)),
        pl.BlockSpec((2, Qn, PACK * Dn), lambda: (0, 0, 0)),
        pl.BlockSpec((Qn, 1), lambda: (0, 0)),
        pl.BlockSpec((Qn, 1), lambda: (0, 0)),
        pl.BlockSpec(memory_space=pltpu.SMEM),
    ],
    out_specs=pl.BlockSpec((Qn, Dn), lambda: (0, 0)),
    out_shape=jax.ShapeDtypeStruct((Qn, Dn), jnp.float32),
)


def kernel(defect_location, defect_sites, ricci_flow_rate, cohesion, friction_angle):
    x = defect_location.astype(jnp.float32)
    sites = defect_sites.astype(jnp.float32)

    pad = jnp.full((KPAD - Kn, Dn), 1e6, jnp.float32)
    sites_p = jnp.concatenate([sites, pad], axis=0)
    xt = x.T

    i1, i2 = _scan(sites_p, xt)
    idx_all = jnp.concatenate([i1.reshape(Qn), i2.reshape(Qn)])

    rows_view = sites_p.reshape(KPAD // PACK, PACK * Dn)
    rows = _make_sc_gather()(rows_view, idx_all // PACK)
    rows2 = rows.reshape(2, Qn, PACK * Dn)

    scal = jnp.stack([
        ricci_flow_rate.astype(jnp.float32),
        cohesion.astype(jnp.float32),
        jnp.tan(friction_angle).astype(jnp.float32),
    ])
    return _epi(x, rows2, i1.reshape(Qn, 1), i2.reshape(Qn, 1), scal)


# speedup vs baseline: 30.9965x; 1.6658x over previous
import functools

import jax
import jax.numpy as jnp
from jax import lax
from jax.experimental import pallas as pl
from jax.experimental.pallas import tpu as pltpu
from jax.experimental.pallas import tpu_sc as plsc

Qn = 1024
Kn = 100000
Dn = 16
KT = 2048
KPAD = 100352
NT = KPAD // KT
PACK = 128 // Dn
NW = 32
BPW = (2 * Qn) // NW

_BIG_I32 = 2**30
_BIG_F32 = 2.0**23


def _merge2(b1v, b1i, b2v, b2i, v, i):
    b1v_o, b1i_o = b1v[...], b1i[...]
    b2v_o, b2i_o = b2v[...], b2i[...]
    better1 = v < b1v_o
    better2 = v < b2v_o
    b1v[...] = jnp.where(better1, v, b1v_o)
    b1i[...] = jnp.where(better1, i, b1i_o)
    b2v[...] = jnp.where(better1, b1v_o, jnp.where(better2, v, b2v_o))
    b2i[...] = jnp.where(better1, b1i_o, jnp.where(better2, i, b2i_o))


def _scan_kernel(s_ref, xt2_ref, i1_ref, i2_ref, b1v, b1i, b2v, b2i):
    k = pl.program_id(0)

    @pl.when(k == 0)
    def _init():
        b1v[...] = jnp.full((1, Qn), jnp.inf, jnp.float32)
        b2v[...] = jnp.full((1, Qn), jnp.inf, jnp.float32)
        b1i[...] = jnp.zeros((1, Qn), jnp.int32)
        b2i[...] = jnp.zeros((1, Qn), jnp.int32)

    s = s_ref[...]
    colnorm = jnp.sum(s * s, axis=1, keepdims=True)
    s_hi = s.astype(jnp.bfloat16)
    s_lo = (s - s_hi.astype(jnp.float32)).astype(jnp.bfloat16)
    lhs = jnp.concatenate([s_hi, s_hi, s_lo], axis=1)
    nxs2 = lax.dot_general(
        lhs, xt2_ref[...], (((1,), (0,)), ((), ())),
        preferred_element_type=jnp.float32)
    scores = colnorm + nxs2

    row = lax.broadcasted_iota(jnp.int32, (KT, Qn), 0).astype(jnp.float32)
    t1 = jnp.min(scores, axis=0, keepdims=True)
    eq1 = scores == t1
    c1 = jnp.min(jnp.where(eq1, row, _BIG_F32),
                 axis=0, keepdims=True).astype(jnp.int32) + k * KT
    scores2 = jnp.where(eq1, jnp.inf, scores)
    t2 = jnp.min(scores2, axis=0, keepdims=True)
    c2 = jnp.min(jnp.where(scores2 == t2, row, _BIG_F32),
                 axis=0, keepdims=True).astype(jnp.int32) + k * KT

    _merge2(b1v, b1i, b2v, b2i, t1, c1)
    _merge2(b1v, b1i, b2v, b2i, t2, c2)

    @pl.when(k == NT - 1)
    def _fin():
        i1_ref[...] = b1i[...]
        i2_ref[...] = b2i[...]


_scan = pl.pallas_call(
    _scan_kernel,
    grid=(NT,),
    in_specs=[
        pl.BlockSpec((KT, Dn), lambda k: (k, 0)),
        pl.BlockSpec((3 * Dn, Qn), lambda k: (0, 0)),
    ],
    out_specs=[
        pl.BlockSpec((1, Qn), lambda k: (0, 0)),
        pl.BlockSpec((1, Qn), lambda k: (0, 0)),
    ],
    out_shape=[
        jax.ShapeDtypeStruct((1, Qn), jnp.int32),
        jax.ShapeDtypeStruct((1, Qn), jnp.int32),
    ],
    scratch_shapes=[
        pltpu.VMEM((1, Qn), jnp.float32),
        pltpu.VMEM((1, Qn), jnp.int32),
        pltpu.VMEM((1, Qn), jnp.float32),
        pltpu.VMEM((1, Qn), jnp.int32),
    ],
)


@functools.cache
def _make_sc_gather():
    @functools.partial(
        pl.kernel,
        mesh=plsc.VectorSubcoreMesh(core_axis_name="c", subcore_axis_name="s"),
        out_type=jax.ShapeDtypeStruct((2 * Qn, 8 * Dn), jnp.float32),
        scratch_types=[
            pltpu.VMEM((BPW,), jnp.int32),
            pltpu.VMEM((BPW, 8 * Dn), jnp.float32),
            pltpu.SemaphoreType.DMA,
        ],
    )
    def _sc_gather(rows_hbm, idx_hbm, out_hbm, idx_v, rows_v, sem):
        wid = lax.axis_index("s") * 2 + lax.axis_index("c")
        base = wid * BPW
        pltpu.sync_copy(idx_hbm.at[pl.ds(base, BPW)], idx_v)
        pltpu.async_copy(rows_hbm.at[idx_v], rows_v, sem).wait()
        pltpu.sync_copy(rows_v, out_hbm.at[pl.ds(base, BPW)])

    return _sc_gather


def _extract(r, sub):
    lane_grp = lax.broadcasted_iota(jnp.int32, (Qn, PACK * Dn), 1) // Dn
    g = jnp.where(lane_grp == sub, r, 0.0)
    acc = g[:, 0:Dn]
    for c in range(1, PACK):
        acc = acc + g[:, c * Dn:(c + 1) * Dn]
    return acc


def _epi_kernel(x_ref, rows_ref, i1_ref, i2_ref, scal_ref, out_ref):
    x = x_ref[...]
    i1 = i1_ref[...]
    i2 = i2_ref[...]
    s1 = _extract(rows_ref[0], i1 % PACK)
    s2 = _extract(rows_ref[1], i2 % PACK)
    rate = scal_ref[0]
    cohesion = scal_ref[1]
    tanfa = scal_ref[2]

    diff1 = x - s1
    diff2 = x - s2
    d1 = jnp.sqrt(jnp.sum(diff1 * diff1, axis=1, keepdims=True))
    d2 = jnp.sqrt(jnp.sum(diff2 * diff2, axis=1, keepdims=True))
    pick1 = (d1 < d2) | ((d1 == d2) & (i1 < i2))
    sw = jnp.where(pick1, s1, s2)

    ricci = rate * (sw - x)
    mag = jnp.sqrt(jnp.sum(ricci * ricci, axis=1, keepdims=True))
    xnorm = jnp.sqrt(jnp.sum(x * x, axis=1, keepdims=True))
    normal = jnp.abs(jnp.sum(x * ricci, axis=1, keepdims=True)) / (xnorm + 1e-8)
    thresh = cohesion + normal * tanfa
    exceeds = mag > thresh
    out_ref[...] = jnp.where(exceeds, ricci * 2.0, ricci * 0.5)


_epi = pl.pallas_call(
    _epi_kernel,
    in_specs=[
        pl.BlockSpec((Qn, Dn), lambda: (0, 0)),
        pl.BlockSpec((2, Qn, PACK * Dn), lambda: (0, 0, 0)),
        pl.BlockSpec((Qn, 1), lambda: (0, 0)),
        pl.BlockSpec((Qn, 1), lambda: (0, 0)),
        pl.BlockSpec(memory_space=pltpu.SMEM),
    ],
    out_specs=pl.BlockSpec((Qn, Dn), lambda: (0, 0)),
    out_shape=jax.ShapeDtypeStruct((Qn, Dn), jnp.float32),
)


def kernel(defect_location, defect_sites, ricci_flow_rate, cohesion, friction_angle):
    x = defect_location.astype(jnp.float32)
    sites = defect_sites.astype(jnp.float32)

    pad = jnp.full((KPAD - Kn, Dn), 1e6, jnp.float32)
    sites_p = jnp.concatenate([sites, pad], axis=0)
    xt2 = x.T + x.T
    xh2 = xt2.astype(jnp.bfloat16)
    xl2 = (xt2 - xh2.astype(jnp.float32)).astype(jnp.bfloat16)
    xcat = jnp.concatenate([-xh2, -xl2, -xh2], axis=0)

    i1, i2 = _scan(sites_p, xcat)
    idx_all = jnp.concatenate([i1.reshape(Qn), i2.reshape(Qn)])

    rows_view = sites_p.reshape(KPAD // PACK, PACK * Dn)
    rows = _make_sc_gather()(rows_view, idx_all // PACK)
    rows2 = rows.reshape(2, Qn, PACK * Dn)

    scal = jnp.stack([
        ricci_flow_rate.astype(jnp.float32),
        cohesion.astype(jnp.float32),
        jnp.tan(friction_angle).astype(jnp.float32),
    ])
    return _epi(x, rows2, i1.reshape(Qn, 1), i2.reshape(Qn, 1), scal)
